# baseline (device time: 79523 ns/iter reference)
import jax
import jax.numpy as jnp
from jax import lax
from jax.experimental import pallas as pl
from jax.experimental.pallas import tpu as pltpu

N_DEV = 4


def kernel(x, Win0, Wout0, Win1, Wout1, Win2, Wout2):
    b, d_in = x.shape
    h_dim = Win0.shape[1]
    d_out = Wout0.shape[1]

    def body(x_ref, win0_ref, wout0_ref, win1_ref, wout1_ref, win2_ref,
             wout2_ref, out_ref, comm_ref, send_sems, recv_sems):
        my = lax.axis_index("i")
        right = lax.rem(my + 1, N_DEV)
        left = lax.rem(my + N_DEV - 1, N_DEV)

        barrier_sem = pltpu.get_barrier_semaphore()
        for nbr in (left, right):
            pl.semaphore_signal(barrier_sem, inc=1, device_id=(nbr,),
                                device_id_type=pl.DeviceIdType.MESH)
        pl.semaphore_wait(barrier_sem, 2)

        def all_reduce(partial_f32):
            comm_ref[0, :, :] = partial_f32.astype(jnp.bfloat16)
            acc = partial_f32
            for h in range(N_DEV - 1):
                rdma = pltpu.make_async_remote_copy(
                    src_ref=comm_ref.at[h],
                    dst_ref=comm_ref.at[h + 1],
                    send_sem=send_sems.at[h],
                    recv_sem=recv_sems.at[h],
                    device_id=(right,),
                    device_id_type=pl.DeviceIdType.MESH,
                )
                rdma.start()
                rdma.wait()
                acc = acc + comm_ref[h + 1, :, :].astype(jnp.float32)
            return acc

        xl = x_ref[...].astype(jnp.bfloat16)
        layers = ((win0_ref, wout0_ref), (win1_ref, wout1_ref),
                  (win2_ref, wout2_ref))
        y = None
        for li, (win_ref, wout_ref) in enumerate(layers):
            partial = jnp.dot(xl, win_ref[...].astype(jnp.bfloat16),
                              preferred_element_type=jnp.float32)
            h_full = all_reduce(partial)
            h_act = jnp.maximum(h_full, 0.0).astype(jnp.bfloat16)
            y = jnp.dot(h_act, wout_ref[...].astype(jnp.bfloat16),
                        preferred_element_type=jnp.float32)
            if li < len(layers) - 1:
                xl = y.astype(jnp.bfloat16)
        out_ref[...] = y

    return pl.pallas_call(
        body,
        out_shape=jax.ShapeDtypeStruct((b, d_out), jnp.float32),
        in_specs=[pl.BlockSpec(memory_space=pltpu.VMEM)] * 7,
        out_specs=pl.BlockSpec(memory_space=pltpu.VMEM),
        scratch_shapes=[
            pltpu.VMEM((N_DEV, b, h_dim), jnp.bfloat16),
            pltpu.SemaphoreType.DMA((N_DEV - 1,)),
            pltpu.SemaphoreType.DMA((N_DEV - 1,)),
        ],
        compiler_params=pltpu.CompilerParams(collective_id=0),
    )(x, Win0, Wout0, Win1, Wout1, Win2, Wout2)


# device time: 40698 ns/iter; 1.9540x vs baseline; 1.9540x over previous
import jax
import jax.numpy as jnp
from jax import lax
from jax.experimental import pallas as pl
from jax.experimental.pallas import tpu as pltpu

N_DEV = 4


def kernel(x, Win0, Wout0, Win1, Wout1, Win2, Wout2):
    b, d_in = x.shape
    h_dim = Win0.shape[1]
    d_out = Wout0.shape[1]
    hh = h_dim // 2

    def body(x_ref, win0_ref, wout0_ref, win1_ref, wout1_ref, win2_ref,
             wout2_ref, out_ref, send_ref, recv_ref, send_sems, recv_sems):
        my = lax.axis_index("i")
        p1 = my + 1 - 2 * lax.rem(my, 2)
        p2 = N_DEV - 1 - my

        barrier_sem = pltpu.get_barrier_semaphore()
        for nbr in (p1, p2):
            pl.semaphore_signal(barrier_sem, inc=1, device_id=(nbr,),
                                device_id_type=pl.DeviceIdType.MESH)
        pl.semaphore_wait(barrier_sem, 2)

        def exchange(slot, data_bf16, partner):
            send_ref[slot, :, :] = data_bf16
            rdma = pltpu.make_async_remote_copy(
                src_ref=send_ref.at[slot],
                dst_ref=recv_ref.at[slot],
                send_sem=send_sems.at[slot],
                recv_sem=recv_sems.at[slot],
                device_id=(partner,),
                device_id_type=pl.DeviceIdType.MESH,
            )
            rdma.start()
            return rdma

        def all_reduce(partial_f32):
            pa = partial_f32[:, :hh]
            pb = partial_f32[:, hh:]
            ra = exchange(0, pa.astype(jnp.bfloat16), p1)
            rb = exchange(1, pb.astype(jnp.bfloat16), p2)
            ra.wait()
            rb.wait()
            acc_a = pa + recv_ref[0, :, :].astype(jnp.float32)
            acc_b = pb + recv_ref[1, :, :].astype(jnp.float32)
            ra = exchange(2, acc_a.astype(jnp.bfloat16), p2)
            rb = exchange(3, acc_b.astype(jnp.bfloat16), p1)
            ra.wait()
            rb.wait()
            tot_a = acc_a + recv_ref[2, :, :].astype(jnp.float32)
            tot_b = acc_b + recv_ref[3, :, :].astype(jnp.float32)
            return jnp.concatenate([tot_a, tot_b], axis=1)

        xl = x_ref[...].astype(jnp.bfloat16)
        layers = ((win0_ref, wout0_ref), (win1_ref, wout1_ref),
                  (win2_ref, wout2_ref))
        y = None
        for li, (win_ref, wout_ref) in enumerate(layers):
            partial = jnp.dot(xl, win_ref[...].astype(jnp.bfloat16),
                              preferred_element_type=jnp.float32)
            h_full = all_reduce(partial)
            h_act = jnp.maximum(h_full, 0.0).astype(jnp.bfloat16)
            y = jnp.dot(h_act, wout_ref[...].astype(jnp.bfloat16),
                        preferred_element_type=jnp.float32)
            if li < len(layers) - 1:
                xl = y.astype(jnp.bfloat16)
        out_ref[...] = y

    return pl.pallas_call(
        body,
        out_shape=jax.ShapeDtypeStruct((b, d_out), jnp.float32),
        in_specs=[pl.BlockSpec(memory_space=pltpu.VMEM)] * 7,
        out_specs=pl.BlockSpec(memory_space=pltpu.VMEM),
        scratch_shapes=[
            pltpu.VMEM((4, b, hh), jnp.bfloat16),
            pltpu.VMEM((4, b, hh), jnp.bfloat16),
            pltpu.SemaphoreType.DMA((4,)),
            pltpu.SemaphoreType.DMA((4,)),
        ],
        compiler_params=pltpu.CompilerParams(collective_id=0),
    )(x, Win0, Wout0, Win1, Wout1, Win2, Wout2)


# device time: 32999 ns/iter; 2.4099x vs baseline; 1.2333x over previous
import jax
import jax.numpy as jnp
from jax import lax
from jax.experimental import pallas as pl
from jax.experimental.pallas import tpu as pltpu

N_DEV = 4
CH = 2
N_LAYERS = 3


def kernel(x, Win0, Wout0, Win1, Wout1, Win2, Wout2):
    b, d_in = x.shape
    h_dim = Win0.shape[1]
    d_out = Wout0.shape[1]
    hh = h_dim // 2
    rows = b // CH

    def body(x_ref, win0_ref, wout0_ref, win1_ref, wout1_ref, win2_ref,
             wout2_ref, out_ref, send_ref, recv_ref, send_sems, recv_sems):
        my = lax.axis_index("i")
        p1 = my + 1 - 2 * lax.rem(my, 2)
        p2 = N_DEV - 1 - my

        barrier_sem = pltpu.get_barrier_semaphore()
        for nbr in (p1, p2):
            pl.semaphore_signal(barrier_sem, inc=1, device_id=(nbr,),
                                device_id_type=pl.DeviceIdType.MESH)
        pl.semaphore_wait(barrier_sem, 2)

        def slot(c, stage, half):
            return c * 4 + stage * 2 + half

        def exchange(sl, data_bf16, partner):
            send_ref[sl, :, :] = data_bf16
            rdma = pltpu.make_async_remote_copy(
                src_ref=send_ref.at[sl],
                dst_ref=recv_ref.at[sl],
                send_sem=send_sems.at[sl],
                recv_sem=recv_sems.at[sl],
                device_id=(partner,),
                device_id_type=pl.DeviceIdType.MESH,
            )
            rdma.start()
            return rdma

        wins = (win0_ref, win1_ref, win2_ref)
        wouts = (wout0_ref, wout1_ref, wout2_ref)

        xl = [x_ref[pl.ds(c * rows, rows), :].astype(jnp.bfloat16)
              for c in range(CH)]
        part = [None] * CH
        acc = [None] * CH
        rd = [None] * CH

        def s1(l, c):
            partial = jnp.dot(xl[c], wins[l][...].astype(jnp.bfloat16),
                              preferred_element_type=jnp.float32)
            pa, pb = partial[:, :hh], partial[:, hh:]
            ra = exchange(slot(c, 0, 0), pa.astype(jnp.bfloat16), p1)
            rb = exchange(slot(c, 0, 1), pb.astype(jnp.bfloat16), p2)
            part[c] = (pa, pb)
            rd[c] = (ra, rb)

        def s2(l, c):
            ra, rb = rd[c]
            ra.wait()
            rb.wait()
            pa, pb = part[c]
            acc_a = pa + recv_ref[slot(c, 0, 0), :, :].astype(jnp.float32)
            acc_b = pb + recv_ref[slot(c, 0, 1), :, :].astype(jnp.float32)
            ra = exchange(slot(c, 1, 0), acc_a.astype(jnp.bfloat16), p2)
            rb = exchange(slot(c, 1, 1), acc_b.astype(jnp.bfloat16), p1)
            acc[c] = (acc_a, acc_b)
            rd[c] = (ra, rb)

        def s3(l, c):
            ra, rb = rd[c]
            ra.wait()
            rb.wait()
            acc_a, acc_b = acc[c]
            tot_a = acc_a + recv_ref[slot(c, 1, 0), :, :].astype(jnp.float32)
            tot_b = acc_b + recv_ref[slot(c, 1, 1), :, :].astype(jnp.float32)
            h_act = jnp.concatenate(
                [jnp.maximum(tot_a, 0.0), jnp.maximum(tot_b, 0.0)],
                axis=1).astype(jnp.bfloat16)
            y = jnp.dot(h_act, wouts[l][...].astype(jnp.bfloat16),
                        preferred_element_type=jnp.float32)
            if l < N_LAYERS - 1:
                xl[c] = y.astype(jnp.bfloat16)
            else:
                out_ref[pl.ds(c * rows, rows), :] = y

        for c in range(CH):
            s1(0, c)
        for l in range(N_LAYERS):
            for c in range(CH):
                s2(l, c)
            for c in range(CH):
                s3(l, c)
                if l < N_LAYERS - 1:
                    s1(l + 1, c)

    n_slots = CH * 4
    return pl.pallas_call(
        body,
        out_shape=jax.ShapeDtypeStruct((b, d_out), jnp.float32),
        in_specs=[pl.BlockSpec(memory_space=pltpu.VMEM)] * 7,
        out_specs=pl.BlockSpec(memory_space=pltpu.VMEM),
        scratch_shapes=[
            pltpu.VMEM((n_slots, rows, hh), jnp.bfloat16),
            pltpu.VMEM((n_slots, rows, hh), jnp.bfloat16),
            pltpu.SemaphoreType.DMA((n_slots,)),
            pltpu.SemaphoreType.DMA((n_slots,)),
        ],
        compiler_params=pltpu.CompilerParams(collective_id=0),
    )(x, Win0, Wout0, Win1, Wout1, Win2, Wout2)


# device time: 30711 ns/iter; 2.5894x vs baseline; 1.0745x over previous
import jax
import jax.numpy as jnp
from jax import lax
from jax.experimental import pallas as pl
from jax.experimental.pallas import tpu as pltpu

N_DEV = 4
CH = 4
N_LAYERS = 3


def kernel(x, Win0, Wout0, Win1, Wout1, Win2, Wout2):
    b, d_in = x.shape
    h_dim = Win0.shape[1]
    d_out = Wout0.shape[1]
    hh = h_dim // 2
    rows = b // CH

    def body(x_ref, win0_ref, wout0_ref, win1_ref, wout1_ref, win2_ref,
             wout2_ref, out_ref, send_ref, recv_ref, send_sems, recv_sems):
        my = lax.axis_index("i")
        p1 = my + 1 - 2 * lax.rem(my, 2)
        p2 = N_DEV - 1 - my

        barrier_sem = pltpu.get_barrier_semaphore()
        for nbr in (p1, p2):
            pl.semaphore_signal(barrier_sem, inc=1, device_id=(nbr,),
                                device_id_type=pl.DeviceIdType.MESH)
        pl.semaphore_wait(barrier_sem, 2)

        def slot(c, stage, half):
            return c * 4 + stage * 2 + half

        def exchange(sl, data_bf16, partner):
            send_ref[sl, :, :] = data_bf16
            rdma = pltpu.make_async_remote_copy(
                src_ref=send_ref.at[sl],
                dst_ref=recv_ref.at[sl],
                send_sem=send_sems.at[sl],
                recv_sem=recv_sems.at[sl],
                device_id=(partner,),
                device_id_type=pl.DeviceIdType.MESH,
            )
            rdma.start()
            return rdma

        wins = (win0_ref, win1_ref, win2_ref)
        wouts = (wout0_ref, wout1_ref, wout2_ref)

        xl = [x_ref[pl.ds(c * rows, rows), :].astype(jnp.bfloat16)
              for c in range(CH)]
        part = [None] * CH
        acc = [None] * CH
        rd = [None] * CH

        def s1(l, c):
            partial = jnp.dot(xl[c], wins[l][...].astype(jnp.bfloat16),
                              preferred_element_type=jnp.float32)
            pa, pb = partial[:, :hh], partial[:, hh:]
            ra = exchange(slot(c, 0, 0), pa.astype(jnp.bfloat16), p1)
            rb = exchange(slot(c, 0, 1), pb.astype(jnp.bfloat16), p2)
            part[c] = (pa, pb)
            rd[c] = (ra, rb)

        def s2(l, c):
            ra, rb = rd[c]
            ra.wait()
            rb.wait()
            pa, pb = part[c]
            acc_a = pa + recv_ref[slot(c, 0, 0), :, :].astype(jnp.float32)
            acc_b = pb + recv_ref[slot(c, 0, 1), :, :].astype(jnp.float32)
            ra = exchange(slot(c, 1, 0), acc_a.astype(jnp.bfloat16), p2)
            rb = exchange(slot(c, 1, 1), acc_b.astype(jnp.bfloat16), p1)
            acc[c] = (acc_a, acc_b)
            rd[c] = (ra, rb)

        def s3(l, c):
            ra, rb = rd[c]
            ra.wait()
            rb.wait()
            acc_a, acc_b = acc[c]
            tot_a = acc_a + recv_ref[slot(c, 1, 0), :, :].astype(jnp.float32)
            tot_b = acc_b + recv_ref[slot(c, 1, 1), :, :].astype(jnp.float32)
            h_act = jnp.concatenate(
                [jnp.maximum(tot_a, 0.0), jnp.maximum(tot_b, 0.0)],
                axis=1).astype(jnp.bfloat16)
            y = jnp.dot(h_act, wouts[l][...].astype(jnp.bfloat16),
                        preferred_element_type=jnp.float32)
            if l < N_LAYERS - 1:
                xl[c] = y.astype(jnp.bfloat16)
            else:
                out_ref[pl.ds(c * rows, rows), :] = y

        for c in range(CH):
            s1(0, c)
        for l in range(N_LAYERS):
            for c in range(CH):
                s2(l, c)
            for c in range(CH):
                s3(l, c)
                if l < N_LAYERS - 1:
                    s1(l + 1, c)

    n_slots = CH * 4
    return pl.pallas_call(
        body,
        out_shape=jax.ShapeDtypeStruct((b, d_out), jnp.float32),
        in_specs=[pl.BlockSpec(memory_space=pltpu.VMEM)] * 7,
        out_specs=pl.BlockSpec(memory_space=pltpu.VMEM),
        scratch_shapes=[
            pltpu.VMEM((n_slots, rows, hh), jnp.bfloat16),
            pltpu.VMEM((n_slots, rows, hh), jnp.bfloat16),
            pltpu.SemaphoreType.DMA((n_slots,)),
            pltpu.SemaphoreType.DMA((n_slots,)),
        ],
        compiler_params=pltpu.CompilerParams(collective_id=0),
    )(x, Win0, Wout0, Win1, Wout1, Win2, Wout2)
